# baseline (speedup 1.0000x reference)
import jax
import jax.numpy as jnp
from jax import lax
from jax.experimental import pallas as pl
from jax.experimental.pallas import tpu as pltpu
from jax.experimental.pallas import tpu_sc as plsc

F32 = jnp.float32
NC, NS = 2, 16
NW = NC * NS
HID = 512
RB = 640
ZPAD = 640
EPS = 1e-5
SC_CH = 40
ATOMS_PER_CHUNK = 4


def _sc_mesh():
    return plsc.VectorSubcoreMesh(core_axis_name="c", subcore_axis_name="s")


def _wid():
    return lax.axis_index("s") * NC + lax.axis_index("c")


def _sc_gather_rows(table, idx3, n_rows, d):
    per_w = idx3.shape[1]
    ch = SC_CH
    nch = per_w // ch

    def body(table_h, idx_h, out_h, idx_v, r0, r1, r2, r3,
             g0, g1, g2, g3, o0, o1, o2, o3):
        w = _wid()
        rows = (r0, r1, r2, r3)
        sg = (g0, g1, g2, g3)
        so = (o0, o1, o2, o3)
        pltpu.sync_copy(idx_h.at[w], idx_v)

        def idx_sl(c):
            return idx_v.at[pl.ds(c * ch, ch)]

        def fire(c, b):
            pltpu.async_copy(table_h.at[idx_sl(c)], rows[b], sg[b])

        def wait_gather(c, b):
            pltpu.make_async_copy(table_h.at[idx_sl(c)], rows[b], sg[b]).wait()

        def out_dst(c):
            return out_h.at[pl.ds(w * per_w + c * ch, ch)]

        fire(0, 0)
        fire(1, 1)

        def slot(c, b):
            wait_gather(c, b)
            pltpu.async_copy(rows[b], out_dst(c), so[b])
            nb = (b + 2) % 4

            @pl.when(c + 2 < nch)
            def _():
                @pl.when(c >= 2)
                def _():
                    pltpu.make_async_copy(rows[nb], out_dst(c - 2), so[nb]).wait()
                fire(c + 2, nb)

        def quad(q, carry):
            for b in range(4):
                slot(q * 4 + b, b)
            return carry

        lax.fori_loop(0, nch // 4, quad, 0)
        for t in range(nch - (nch % 4), nch):
            slot(t, t % 4)
        for c in range(nch - 4, nch):
            pltpu.make_async_copy(rows[c % 4], out_dst(c), so[c % 4]).wait()

    return pl.kernel(
        body,
        out_type=jax.ShapeDtypeStruct((n_rows, d), F32),
        mesh=_sc_mesh(),
        scratch_types=[
            pltpu.VMEM((nch * ch,), jnp.int32),
            pltpu.VMEM((ch, d), F32),
            pltpu.VMEM((ch, d), F32),
            pltpu.VMEM((ch, d), F32),
            pltpu.VMEM((ch, d), F32),
            pltpu.SemaphoreType.DMA,
            pltpu.SemaphoreType.DMA,
            pltpu.SemaphoreType.DMA,
            pltpu.SemaphoreType.DMA,
            pltpu.SemaphoreType.DMA,
            pltpu.SemaphoreType.DMA,
            pltpu.SemaphoreType.DMA,
            pltpu.SemaphoreType.DMA,
        ],
    )(table, idx3)


def _sc_atom_sums(gext, gidx3, n_out):
    ch = ATOMS_PER_CHUNK * 16
    nch = gidx3.shape[1] // ch
    apc = ATOMS_PER_CHUNK

    def body(g_h, idx_h, out_h, idx_v, rows0, rows1, acc0, acc1,
             sg0, sg1, so0, so1):
        w = _wid()
        rows = (rows0, rows1)
        acc = (acc0, acc1)
        sg = (sg0, sg1)
        so = (so0, so1)
        pltpu.sync_copy(idx_h.at[w], idx_v)

        def idx_sl(c):
            return idx_v.at[pl.ds(c * ch, ch)]

        def fire(c, b):
            pltpu.async_copy(g_h.at[idx_sl(c)], rows[b], sg[b])

        def out_dst(c):
            return out_h.at[pl.ds(w * nch * apc + c * apc, apc)]

        fire(0, 0)
        fire(1, 1)

        def slot(c, b):
            pltpu.make_async_copy(g_h.at[idx_sl(c)], rows[b], sg[b]).wait()

            @pl.when(c >= 2)
            def _():
                pltpu.make_async_copy(acc[b], out_dst(c - 2), so[b]).wait()

            for g in range(apc):
                def col(j, cc):
                    sl = pl.ds(j * 16, 16)
                    s = rows[b][g * 16, sl]
                    for k in range(1, 16):
                        s = s + rows[b][g * 16 + k, sl]
                    acc[b][g, sl] = s
                    return cc
                lax.fori_loop(0, HID // 16, col, 0)

            @pl.when(c + 2 < nch)
            def _():
                fire(c + 2, b)

            pltpu.async_copy(acc[b], out_dst(c), so[b])

        def pair(p, carry):
            for b in range(2):
                slot(p * 2 + b, b)
            return carry

        lax.fori_loop(0, nch // 2, pair, 0)
        if nch % 2:
            slot(nch - 1, 0)
        for c in range(nch - 2, nch):
            pltpu.make_async_copy(acc[c % 2], out_dst(c), so[c % 2]).wait()

    return pl.kernel(
        body,
        out_type=jax.ShapeDtypeStruct((n_out, HID), F32),
        mesh=_sc_mesh(),
        scratch_types=[
            pltpu.VMEM((nch * ch,), jnp.int32),
            pltpu.VMEM((ch, HID), F32),
            pltpu.VMEM((ch, HID), F32),
            pltpu.VMEM((apc, HID), F32),
            pltpu.VMEM((apc, HID), F32),
            pltpu.SemaphoreType.DMA,
            pltpu.SemaphoreType.DMA,
            pltpu.SemaphoreType.DMA,
            pltpu.SemaphoreType.DMA,
        ],
    )(gext, gidx3)


def _sc_messages(a_tab, gext, orig3, brm3, n_bonds):
    per_w = orig3.shape[1]
    ch = SC_CH
    nch = per_w // ch

    def body(a_h, g_h, ia_h, ib_h, y_h, ia_v, ib_v,
             ba0, ba1, bb0, bb1, ya_v, sa0, sa1, sb0, sb1, so):
        w = _wid()
        ba = (ba0, ba1)
        bb = (bb0, bb1)
        sa = (sa0, sa1)
        sb = (sb0, sb1)
        pltpu.sync_copy(ia_h.at[w], ia_v)
        pltpu.sync_copy(ib_h.at[w], ib_v)

        def ia_sl(c):
            return ia_v.at[pl.ds(c * ch, ch)]

        def ib_sl(c):
            return ib_v.at[pl.ds(c * ch, ch)]

        def fire(c, b):
            pltpu.async_copy(a_h.at[ia_sl(c)], ba[b], sa[b])
            pltpu.async_copy(g_h.at[ib_sl(c)], bb[b], sb[b])

        def out_dst(c):
            return y_h.at[pl.ds(w * per_w + c * ch, ch)]

        fire(0, 0)
        fire(1, 1)

        def slot(c, b):
            pltpu.make_async_copy(a_h.at[ia_sl(c)], ba[b], sa[b]).wait()
            pltpu.make_async_copy(g_h.at[ib_sl(c)], bb[b], sb[b]).wait()

            @pl.when(c >= 1)
            def _():
                pltpu.make_async_copy(ya_v, out_dst(c - 1), so).wait()

            def row(r, cc):
                for j in range(HID // 16):
                    sl = pl.ds(j * 16, 16)
                    ya_v[r, sl] = ba[b][r, sl] - bb[b][r, sl]
                return cc

            lax.fori_loop(0, ch, row, 0)

            @pl.when(c + 2 < nch)
            def _():
                fire(c + 2, b)

            pltpu.async_copy(ya_v, out_dst(c), so)

        def pair(p, carry):
            for b in range(2):
                slot(p * 2 + b, b)
            return carry

        lax.fori_loop(0, nch // 2, pair, 0)
        if nch % 2:
            slot(nch - 1, 0)
        pltpu.make_async_copy(ya_v, out_dst(nch - 1), so).wait()

    return pl.kernel(
        body,
        out_type=jax.ShapeDtypeStruct((n_bonds, HID), F32),
        mesh=_sc_mesh(),
        scratch_types=[
            pltpu.VMEM((nch * ch,), jnp.int32),
            pltpu.VMEM((nch * ch,), jnp.int32),
            pltpu.VMEM((ch, HID), F32),
            pltpu.VMEM((ch, HID), F32),
            pltpu.VMEM((ch, HID), F32),
            pltpu.VMEM((ch, HID), F32),
            pltpu.VMEM((ch, HID), F32),
            pltpu.SemaphoreType.DMA,
            pltpu.SemaphoreType.DMA,
            pltpu.SemaphoreType.DMA,
            pltpu.SemaphoreType.DMA,
            pltpu.SemaphoreType.DMA,
        ],
    )(a_tab, gext, orig3, brm3)


def _stats_block(y):
    s = jnp.sum(y, axis=0, keepdims=True)
    q = jnp.sum(y * y, axis=0, keepdims=True)
    return jnp.concatenate([s, q, jnp.zeros((6, HID), F32)], axis=0)


def _tc_h0(bf, ex, w_bond, w_atom):
    n = bf.shape[0]
    nb = n // RB
    bdim, adim = bf.shape[1], ex.shape[1]

    def body(bf_ref, ex_ref, wb_ref, wa_ref, out_ref, st_ref):
        i = pl.program_id(0)
        y = jnp.dot(bf_ref[...], wb_ref[...], preferred_element_type=F32)
        y = y + jnp.dot(ex_ref[...], wa_ref[...], preferred_element_type=F32)
        out_ref[...] = y.astype(jnp.bfloat16)

        @pl.when(i == 0)
        def _():
            st_ref[...] = jnp.zeros_like(st_ref)

        st_ref[...] += _stats_block(y)

    return pl.pallas_call(
        body,
        grid=(nb,),
        in_specs=[
            pl.BlockSpec((RB, bdim), lambda i: (i, 0)),
            pl.BlockSpec((RB, adim), lambda i: (i, 0)),
            pl.BlockSpec((bdim, HID), lambda i: (0, 0)),
            pl.BlockSpec((adim, HID), lambda i: (0, 0)),
        ],
        out_specs=[
            pl.BlockSpec((RB, HID), lambda i: (i, 0)),
            pl.BlockSpec((8, HID), lambda i: (0, 0)),
        ],
        out_shape=[
            jax.ShapeDtypeStruct((n, HID), jnp.bfloat16),
            jax.ShapeDtypeStruct((8, HID), F32),
        ],
    )(bf, ex, w_bond, w_atom)


def _tc_msg_matmul(h0p, y, si, bi, sm, bm, w_m):
    n = h0p.shape[0]
    nb = n // RB
    has_y = y is not None

    def body(*refs):
        if has_y:
            h0_ref, y_ref, si_ref, bi_ref, sm_ref, bm_ref, w_ref, out_ref = refs
        else:
            h0_ref, si_ref, bi_ref, w_ref, out_ref = refs
        i = pl.program_id(0)

        @pl.when(i == 0)
        def _():
            out_ref[...] = jnp.zeros_like(out_ref)

        @pl.when(i > 0)
        def _():
            h0 = h0_ref[...].astype(F32)
            h = jnp.maximum(h0 * si_ref[...] + bi_ref[...], 0.0)
            if has_y:
                h = jnp.maximum(h + y_ref[...] * sm_ref[...] + bm_ref[...], 0.0)
            out_ref[...] = jnp.dot(h, w_ref[...], preferred_element_type=F32)

    prev = lambda i: (jnp.maximum(i - 1, 0), 0)
    vec = pl.BlockSpec((1, HID), lambda i: (0, 0))
    in_specs = [pl.BlockSpec((RB, HID), prev)]
    args = [h0p]
    if has_y:
        in_specs.append(pl.BlockSpec((RB, HID), prev))
        args.append(y)
    in_specs += [vec, vec]
    args += [si, bi]
    if has_y:
        in_specs += [vec, vec]
        args += [sm, bm]
    in_specs.append(pl.BlockSpec((HID, HID), lambda i: (0, 0)))
    args.append(w_m)

    return pl.pallas_call(
        body,
        grid=(nb + 1,),
        in_specs=in_specs,
        out_specs=pl.BlockSpec((RB, HID), lambda i: (i, 0)),
        out_shape=jax.ShapeDtypeStruct((n + ZPAD, HID), F32),
    )(*args)


def _tc_stats(y):
    n = y.shape[0]
    nb = n // RB

    def body(y_ref, st_ref):
        i = pl.program_id(0)

        @pl.when(i == 0)
        def _():
            st_ref[...] = jnp.zeros_like(st_ref)

        st_ref[...] += _stats_block(y_ref[...].astype(F32))

    return pl.pallas_call(
        body,
        grid=(nb,),
        in_specs=[pl.BlockSpec((RB, HID), lambda i: (i, 0))],
        out_specs=pl.BlockSpec((8, HID), lambda i: (0, 0)),
        out_shape=jax.ShapeDtypeStruct((8, HID), F32),
    )(y)


def _tc_mv(h0p, y, si, bi, sm, bm, n_atoms):
    n = h0p.shape[0]
    nb = n // RB
    apb = RB // 16

    def body(h0_ref, y_ref, si_ref, bi_ref, sm_ref, bm_ref, out_ref):
        h = jnp.maximum(h0_ref[...].astype(F32) * si_ref[...] + bi_ref[...], 0.0)
        h = jnp.maximum(h + y_ref[...].astype(F32) * sm_ref[...] + bm_ref[...], 0.0)
        out_ref[...] = jnp.sum(h.reshape(apb, 16, HID), axis=1)

    vec = pl.BlockSpec((1, HID), lambda i: (0, 0))
    return pl.pallas_call(
        body,
        grid=(nb,),
        in_specs=[
            pl.BlockSpec((RB, HID), lambda i: (i, 0)),
            pl.BlockSpec((RB, HID), lambda i: (i, 0)),
            vec, vec, vec, vec,
        ],
        out_specs=pl.BlockSpec((apb, HID), lambda i: (i, 0)),
        out_shape=jax.ShapeDtypeStruct((n_atoms, HID), F32),
    )(h0p, y, si, bi, sm, bm)


def _tc_atom(af, mv, w_a1, w_a2):
    n, adim = af.shape
    blk = 400
    nb = n // blk

    def body(af_ref, mv_ref, w1_ref, w2_ref, out_ref, st_ref):
        i = pl.program_id(0)
        y = jnp.dot(af_ref[...], w1_ref[...], preferred_element_type=F32)
        y = y + jnp.dot(mv_ref[...], w2_ref[...], preferred_element_type=F32)
        out_ref[...] = y

        @pl.when(i == 0)
        def _():
            st_ref[...] = jnp.zeros_like(st_ref)

        st_ref[...] += _stats_block(y)

    return pl.pallas_call(
        body,
        grid=(nb,),
        in_specs=[
            pl.BlockSpec((blk, adim), lambda i: (i, 0)),
            pl.BlockSpec((blk, HID), lambda i: (i, 0)),
            pl.BlockSpec((adim, HID), lambda i: (0, 0)),
            pl.BlockSpec((HID, HID), lambda i: (0, 0)),
        ],
        out_specs=[
            pl.BlockSpec((blk, HID), lambda i: (i, 0)),
            pl.BlockSpec((8, HID), lambda i: (0, 0)),
        ],
        out_shape=[
            jax.ShapeDtypeStruct((n, HID), F32),
            jax.ShapeDtypeStruct((8, HID), F32),
        ],
    )(af, mv, w_a1, w_a2)


def _tc_mol(pre, sa, ba, n_mol, atoms_per_mol):
    n = pre.shape[0]
    blk_mols = 40
    blk = blk_mols * atoms_per_mol
    nb = n // blk
    inv = 1.0 / atoms_per_mol

    def body(p_ref, sa_ref, ba_ref, out_ref):
        h = jnp.maximum(p_ref[...] * sa_ref[...] + ba_ref[...], 0.0)
        out_ref[...] = jnp.sum(h.reshape(blk_mols, atoms_per_mol, HID), axis=1) * inv

    vec = pl.BlockSpec((1, HID), lambda i: (0, 0))
    return pl.pallas_call(
        body,
        grid=(nb,),
        in_specs=[pl.BlockSpec((blk, HID), lambda i: (i, 0)), vec, vec],
        out_specs=pl.BlockSpec((blk_mols, HID), lambda i: (i, 0)),
        out_shape=jax.ShapeDtypeStruct((n_mol, HID), F32),
    )(pre, sa, ba)


def _scale_shift(st, n, g, b):
    mu = st[0] / n
    var = st[1] / n - mu * mu
    sc = g * lax.rsqrt(var + EPS)
    return sc.reshape(1, HID), (b - mu * sc).reshape(1, HID)


def kernel(atom_features, bond_features, bond_origins, molecule_features,
           atom_incoming_bond_map, bond_reverse_map, num_bonds_per_atom,
           num_atoms_per_mol, W_i, W_m, W_a,
           bn_i_g, bn_i_b, bn_m_g, bn_m_b, bn_a_g, bn_a_b):
    n_atoms, adim = atom_features.shape
    n_bonds, bdim = bond_features.shape
    n_mol = molecule_features.shape[0]

    orig = bond_origins.astype(jnp.int32)
    orig3 = orig.reshape(NW, -1)
    brm3 = (bond_reverse_map.astype(jnp.int32) + ZPAD).reshape(NW, -1)

    apw = -(-n_atoms // (NW * ATOMS_PER_CHUNK)) * ATOMS_PER_CHUNK
    a_pad = NW * apw
    aibm = atom_incoming_bond_map.astype(jnp.int32)
    gidx = jnp.where(aibm == 0, 0, aibm + (ZPAD - 1))
    gidx = jnp.concatenate(
        [gidx, jnp.zeros((a_pad - n_atoms, gidx.shape[1]), jnp.int32)], axis=0)
    gidx3 = gidx.reshape(NW, apw * 16)

    ex = _sc_gather_rows(atom_features, orig3, n_bonds, adim)
    h0p, st0 = _tc_h0(bond_features, ex, W_i[:bdim], W_i[bdim:])
    si, bi = _scale_shift(st0, n_bonds, bn_i_g, bn_i_b)

    y = None
    sm = bm = None
    for _ in range(3):
        gext = _tc_msg_matmul(h0p, y, si, bi, sm, bm, W_m)
        a_tab = _sc_atom_sums(gext, gidx3, a_pad)
        y = _sc_messages(a_tab, gext, orig3, brm3, n_bonds)
        st = _tc_stats(y)
        sm, bm = _scale_shift(st, n_bonds, bn_m_g, bn_m_b)

    mv = _tc_mv(h0p, y, si, bi, sm, bm, n_atoms)
    mv = mv + (jnp.asarray(num_bonds_per_atom) - n_bonds // n_atoms).astype(F32)
    pre, sta = _tc_atom(atom_features, mv, W_a[:adim], W_a[adim:])
    sa, ba = _scale_shift(sta, n_atoms, bn_a_g, bn_a_b)
    h = _tc_mol(pre, sa, ba, n_mol, n_atoms // n_mol)
    h = h + (jnp.asarray(num_atoms_per_mol) - n_atoms // n_mol).astype(F32)
    return jnp.concatenate([h, molecule_features], axis=1)


# speedup vs baseline: 1.0366x; 1.0366x over previous
import jax
import jax.numpy as jnp
from jax import lax
from jax.experimental import pallas as pl
from jax.experimental.pallas import tpu as pltpu
from jax.experimental.pallas import tpu_sc as plsc

F32 = jnp.float32
NC, NS = 2, 16
NW = NC * NS
HID = 512
RB = 640
ZPAD = 640
EPS = 1e-5
SC_CH = 40
ATOMS_PER_CHUNK = 4


def _sc_mesh():
    return plsc.VectorSubcoreMesh(core_axis_name="c", subcore_axis_name="s")


def _wid():
    return lax.axis_index("s") * NC + lax.axis_index("c")


def _sc_gather_rows(table, idx3, n_rows, d):
    per_w = idx3.shape[1]
    ch = SC_CH
    nch = per_w // ch

    def body(table_h, idx_h, out_h, idx_v, r0, r1, r2, r3,
             g0, g1, g2, g3, o0, o1, o2, o3):
        w = _wid()
        rows = (r0, r1, r2, r3)
        sg = (g0, g1, g2, g3)
        so = (o0, o1, o2, o3)
        pltpu.sync_copy(idx_h.at[w], idx_v)

        def idx_sl(c):
            return idx_v.at[pl.ds(c * ch, ch)]

        def fire(c, b):
            pltpu.async_copy(table_h.at[idx_sl(c)], rows[b], sg[b])

        def wait_gather(c, b):
            pltpu.make_async_copy(table_h.at[idx_sl(c)], rows[b], sg[b]).wait()

        def out_dst(c):
            return out_h.at[pl.ds(w * per_w + c * ch, ch)]

        fire(0, 0)
        fire(1, 1)

        def slot(c, b):
            wait_gather(c, b)
            pltpu.async_copy(rows[b], out_dst(c), so[b])
            nb = (b + 2) % 4

            @pl.when(c + 2 < nch)
            def _():
                @pl.when(c >= 2)
                def _():
                    pltpu.make_async_copy(rows[nb], out_dst(c - 2), so[nb]).wait()
                fire(c + 2, nb)

        def quad(q, carry):
            for b in range(4):
                slot(q * 4 + b, b)
            return carry

        lax.fori_loop(0, nch // 4, quad, 0)
        for t in range(nch - (nch % 4), nch):
            slot(t, t % 4)
        for c in range(nch - 4, nch):
            pltpu.make_async_copy(rows[c % 4], out_dst(c), so[c % 4]).wait()

    return pl.kernel(
        body,
        out_type=jax.ShapeDtypeStruct((n_rows, d), F32),
        mesh=_sc_mesh(),
        scratch_types=[
            pltpu.VMEM((nch * ch,), jnp.int32),
            pltpu.VMEM((ch, d), F32),
            pltpu.VMEM((ch, d), F32),
            pltpu.VMEM((ch, d), F32),
            pltpu.VMEM((ch, d), F32),
            pltpu.SemaphoreType.DMA,
            pltpu.SemaphoreType.DMA,
            pltpu.SemaphoreType.DMA,
            pltpu.SemaphoreType.DMA,
            pltpu.SemaphoreType.DMA,
            pltpu.SemaphoreType.DMA,
            pltpu.SemaphoreType.DMA,
            pltpu.SemaphoreType.DMA,
        ],
    )(table, idx3)


def _sc_atom_sums(gext, gidx3, n_out):
    ch = ATOMS_PER_CHUNK * 16
    nch = gidx3.shape[1] // ch
    apc = ATOMS_PER_CHUNK

    def body(g_h, idx_h, out_h, idx_v, rows0, rows1, acc0, acc1,
             sg0, sg1, so0, so1):
        w = _wid()
        rows = (rows0, rows1)
        acc = (acc0, acc1)
        sg = (sg0, sg1)
        so = (so0, so1)
        pltpu.sync_copy(idx_h.at[w], idx_v)

        def idx_sl(c):
            return idx_v.at[pl.ds(c * ch, ch)]

        def fire(c, b):
            pltpu.async_copy(g_h.at[idx_sl(c)], rows[b], sg[b])

        def out_dst(c):
            return out_h.at[pl.ds(w * nch * apc + c * apc, apc)]

        fire(0, 0)
        fire(1, 1)

        def slot(c, b):
            pltpu.make_async_copy(g_h.at[idx_sl(c)], rows[b], sg[b]).wait()

            @pl.when(c >= 2)
            def _():
                pltpu.make_async_copy(acc[b], out_dst(c - 2), so[b]).wait()

            for g in range(apc):
                def col(j, cc):
                    sl = pl.ds(j * 16, 16)
                    s = rows[b][g * 16, sl]
                    for k in range(1, 16):
                        s = s + rows[b][g * 16 + k, sl]
                    acc[b][g, sl] = s
                    return cc
                lax.fori_loop(0, HID // 16, col, 0)

            @pl.when(c + 2 < nch)
            def _():
                fire(c + 2, b)

            pltpu.async_copy(acc[b], out_dst(c), so[b])

        def pair(p, carry):
            for b in range(2):
                slot(p * 2 + b, b)
            return carry

        lax.fori_loop(0, nch // 2, pair, 0)
        if nch % 2:
            slot(nch - 1, 0)
        for c in range(nch - 2, nch):
            pltpu.make_async_copy(acc[c % 2], out_dst(c), so[c % 2]).wait()

    return pl.kernel(
        body,
        out_type=jax.ShapeDtypeStruct((n_out, HID), F32),
        mesh=_sc_mesh(),
        scratch_types=[
            pltpu.VMEM((nch * ch,), jnp.int32),
            pltpu.VMEM((ch, HID), F32),
            pltpu.VMEM((ch, HID), F32),
            pltpu.VMEM((apc, HID), F32),
            pltpu.VMEM((apc, HID), F32),
            pltpu.SemaphoreType.DMA,
            pltpu.SemaphoreType.DMA,
            pltpu.SemaphoreType.DMA,
            pltpu.SemaphoreType.DMA,
        ],
    )(gext, gidx3)


def _sc_messages(a_tab, gext, orig3, brm3, n_bonds):
    per_w = orig3.shape[1]
    ch = SC_CH
    nch = per_w // ch

    def body(a_h, g_h, ia_h, ib_h, y_h, ia_v, ib_v,
             ba0, ba1, bb0, bb1, ya_v, sa0, sa1, sb0, sb1, so):
        w = _wid()
        ba = (ba0, ba1)
        bb = (bb0, bb1)
        sa = (sa0, sa1)
        sb = (sb0, sb1)
        pltpu.sync_copy(ia_h.at[w], ia_v)
        pltpu.sync_copy(ib_h.at[w], ib_v)

        def ia_sl(c):
            return ia_v.at[pl.ds(c * ch, ch)]

        def ib_sl(c):
            return ib_v.at[pl.ds(c * ch, ch)]

        def fire(c, b):
            pltpu.async_copy(a_h.at[ia_sl(c)], ba[b], sa[b])
            pltpu.async_copy(g_h.at[ib_sl(c)], bb[b], sb[b])

        def out_dst(c):
            return y_h.at[pl.ds(w * per_w + c * ch, ch)]

        fire(0, 0)
        fire(1, 1)

        def slot(c, b):
            pltpu.make_async_copy(a_h.at[ia_sl(c)], ba[b], sa[b]).wait()
            pltpu.make_async_copy(g_h.at[ib_sl(c)], bb[b], sb[b]).wait()

            @pl.when(c >= 1)
            def _():
                pltpu.make_async_copy(ya_v, out_dst(c - 1), so).wait()

            def row(r, cc):
                for j in range(HID // 16):
                    sl = pl.ds(j * 16, 16)
                    ya_v[r, sl] = ba[b][r, sl] - bb[b][r, sl]
                return cc

            lax.fori_loop(0, ch, row, 0)

            @pl.when(c + 2 < nch)
            def _():
                fire(c + 2, b)

            pltpu.async_copy(ya_v, out_dst(c), so)

        def pair(p, carry):
            for b in range(2):
                slot(p * 2 + b, b)
            return carry

        lax.fori_loop(0, nch // 2, pair, 0)
        if nch % 2:
            slot(nch - 1, 0)
        pltpu.make_async_copy(ya_v, out_dst(nch - 1), so).wait()

    return pl.kernel(
        body,
        out_type=jax.ShapeDtypeStruct((n_bonds, HID), F32),
        mesh=_sc_mesh(),
        scratch_types=[
            pltpu.VMEM((nch * ch,), jnp.int32),
            pltpu.VMEM((nch * ch,), jnp.int32),
            pltpu.VMEM((ch, HID), F32),
            pltpu.VMEM((ch, HID), F32),
            pltpu.VMEM((ch, HID), F32),
            pltpu.VMEM((ch, HID), F32),
            pltpu.VMEM((ch, HID), F32),
            pltpu.SemaphoreType.DMA,
            pltpu.SemaphoreType.DMA,
            pltpu.SemaphoreType.DMA,
            pltpu.SemaphoreType.DMA,
            pltpu.SemaphoreType.DMA,
        ],
    )(a_tab, gext, orig3, brm3)


def _stats_block(y):
    s = jnp.sum(y, axis=0, keepdims=True)
    q = jnp.sum(y * y, axis=0, keepdims=True)
    return jnp.concatenate([s, q, jnp.zeros((6, HID), F32)], axis=0)


def _tc_h0(bf, ex, w_bond, w_atom):
    n = bf.shape[0]
    nb = n // RB
    bdim, adim = bf.shape[1], ex.shape[1]

    def body(bf_ref, ex_ref, wb_ref, wa_ref, out_ref, st_ref):
        i = pl.program_id(0)
        y = jnp.dot(bf_ref[...], wb_ref[...], preferred_element_type=F32)
        y = y + jnp.dot(ex_ref[...], wa_ref[...], preferred_element_type=F32)
        out_ref[...] = y.astype(jnp.bfloat16)

        @pl.when(i == 0)
        def _():
            st_ref[...] = jnp.zeros_like(st_ref)

        st_ref[...] += _stats_block(y)

    return pl.pallas_call(
        body,
        grid=(nb,),
        in_specs=[
            pl.BlockSpec((RB, bdim), lambda i: (i, 0)),
            pl.BlockSpec((RB, adim), lambda i: (i, 0)),
            pl.BlockSpec((bdim, HID), lambda i: (0, 0)),
            pl.BlockSpec((adim, HID), lambda i: (0, 0)),
        ],
        out_specs=[
            pl.BlockSpec((RB, HID), lambda i: (i, 0)),
            pl.BlockSpec((8, HID), lambda i: (0, 0)),
        ],
        out_shape=[
            jax.ShapeDtypeStruct((n, HID), jnp.bfloat16),
            jax.ShapeDtypeStruct((8, HID), F32),
        ],
    )(bf, ex, w_bond, w_atom)


def _tc_msg_matmul(h0p, y, si, bi, sm, bm, w_m):
    n = h0p.shape[0]
    nb = n // RB
    has_y = y is not None

    def body(*refs):
        if has_y:
            h0_ref, y_ref, si_ref, bi_ref, sm_ref, bm_ref, w_ref, out_ref = refs
        else:
            h0_ref, si_ref, bi_ref, w_ref, out_ref = refs
        i = pl.program_id(0)

        @pl.when(i == 0)
        def _():
            out_ref[...] = jnp.zeros_like(out_ref)

        @pl.when(i > 0)
        def _():
            h0 = h0_ref[...].astype(F32)
            h = jnp.maximum(h0 * si_ref[...] + bi_ref[...], 0.0)
            if has_y:
                h = jnp.maximum(h + y_ref[...] * sm_ref[...] + bm_ref[...], 0.0)
            out_ref[...] = jnp.dot(h, w_ref[...], preferred_element_type=F32)

    prev = lambda i: (jnp.maximum(i - 1, 0), 0)
    vec = pl.BlockSpec((1, HID), lambda i: (0, 0))
    in_specs = [pl.BlockSpec((RB, HID), prev)]
    args = [h0p]
    if has_y:
        in_specs.append(pl.BlockSpec((RB, HID), prev))
        args.append(y)
    in_specs += [vec, vec]
    args += [si, bi]
    if has_y:
        in_specs += [vec, vec]
        args += [sm, bm]
    in_specs.append(pl.BlockSpec((HID, HID), lambda i: (0, 0)))
    args.append(w_m)

    return pl.pallas_call(
        body,
        grid=(nb + 1,),
        in_specs=in_specs,
        out_specs=pl.BlockSpec((RB, HID), lambda i: (i, 0)),
        out_shape=jax.ShapeDtypeStruct((n + ZPAD, HID), F32),
    )(*args)


def _tc_stats(y):
    n = y.shape[0]
    nb = n // RB

    def body(y_ref, st_ref):
        i = pl.program_id(0)

        @pl.when(i == 0)
        def _():
            st_ref[...] = jnp.zeros_like(st_ref)

        st_ref[...] += _stats_block(y_ref[...].astype(F32))

    return pl.pallas_call(
        body,
        grid=(nb,),
        in_specs=[pl.BlockSpec((RB, HID), lambda i: (i, 0))],
        out_specs=pl.BlockSpec((8, HID), lambda i: (0, 0)),
        out_shape=jax.ShapeDtypeStruct((8, HID), F32),
    )(y)


def _tc_readout(h0p, y, si, bi, sm, bm, af, w_a1, w_a2, cvec):
    n, adim = af.shape
    blk = 400
    nb = n // blk
    bpb = blk * 16

    def body(h0_ref, y_ref, si_ref, bi_ref, sm_ref, bm_ref,
             af_ref, w1_ref, w2_ref, cv_ref, out_ref, st_ref):
        i = pl.program_id(0)
        h = jnp.maximum(h0_ref[...].astype(F32) * si_ref[...] + bi_ref[...], 0.0)
        h = jnp.maximum(h + y_ref[...] * sm_ref[...] + bm_ref[...], 0.0)
        m = jnp.sum(h.reshape(blk, 16, HID), axis=1)
        p = jnp.dot(af_ref[...], w1_ref[...], preferred_element_type=F32)
        p = p + jnp.dot(m, w2_ref[...], preferred_element_type=F32) + cv_ref[...]
        out_ref[...] = p

        @pl.when(i == 0)
        def _():
            st_ref[...] = jnp.zeros_like(st_ref)

        st_ref[...] += _stats_block(p)

    vec = pl.BlockSpec((1, HID), lambda i: (0, 0))
    return pl.pallas_call(
        body,
        grid=(nb,),
        in_specs=[
            pl.BlockSpec((bpb, HID), lambda i: (i, 0)),
            pl.BlockSpec((bpb, HID), lambda i: (i, 0)),
            vec, vec, vec, vec,
            pl.BlockSpec((blk, adim), lambda i: (i, 0)),
            pl.BlockSpec((adim, HID), lambda i: (0, 0)),
            pl.BlockSpec((HID, HID), lambda i: (0, 0)),
            vec,
        ],
        out_specs=[
            pl.BlockSpec((blk, HID), lambda i: (i, 0)),
            pl.BlockSpec((8, HID), lambda i: (0, 0)),
        ],
        out_shape=[
            jax.ShapeDtypeStruct((n, HID), F32),
            jax.ShapeDtypeStruct((8, HID), F32),
        ],
    )(h0p, y, si, bi, sm, bm, af, w_a1, w_a2, cvec)


def _tc_mol(pre, sa, ba, n_mol, atoms_per_mol):
    n = pre.shape[0]
    blk_mols = 40
    blk = blk_mols * atoms_per_mol
    nb = n // blk
    inv = 1.0 / atoms_per_mol

    def body(p_ref, sa_ref, ba_ref, out_ref):
        h = jnp.maximum(p_ref[...] * sa_ref[...] + ba_ref[...], 0.0)
        out_ref[...] = jnp.sum(h.reshape(blk_mols, atoms_per_mol, HID), axis=1) * inv

    vec = pl.BlockSpec((1, HID), lambda i: (0, 0))
    return pl.pallas_call(
        body,
        grid=(nb,),
        in_specs=[pl.BlockSpec((blk, HID), lambda i: (i, 0)), vec, vec],
        out_specs=pl.BlockSpec((blk_mols, HID), lambda i: (i, 0)),
        out_shape=jax.ShapeDtypeStruct((n_mol, HID), F32),
    )(pre, sa, ba)


def _scale_shift(st, n, g, b):
    mu = st[0] / n
    var = st[1] / n - mu * mu
    sc = g * lax.rsqrt(var + EPS)
    return sc.reshape(1, HID), (b - mu * sc).reshape(1, HID)


def kernel(atom_features, bond_features, bond_origins, molecule_features,
           atom_incoming_bond_map, bond_reverse_map, num_bonds_per_atom,
           num_atoms_per_mol, W_i, W_m, W_a,
           bn_i_g, bn_i_b, bn_m_g, bn_m_b, bn_a_g, bn_a_b):
    n_atoms, adim = atom_features.shape
    n_bonds, bdim = bond_features.shape
    n_mol = molecule_features.shape[0]

    orig = bond_origins.astype(jnp.int32)
    orig3 = orig.reshape(NW, -1)
    brm3 = (bond_reverse_map.astype(jnp.int32) + ZPAD).reshape(NW, -1)

    apw = -(-n_atoms // (NW * ATOMS_PER_CHUNK)) * ATOMS_PER_CHUNK
    a_pad = NW * apw
    aibm = atom_incoming_bond_map.astype(jnp.int32)
    gidx = jnp.where(aibm == 0, 0, aibm + (ZPAD - 1))
    gidx = jnp.concatenate(
        [gidx, jnp.zeros((a_pad - n_atoms, gidx.shape[1]), jnp.int32)], axis=0)
    gidx3 = gidx.reshape(NW, apw * 16)

    ex = _sc_gather_rows(atom_features, orig3, n_bonds, adim)
    h0p, st0 = _tc_h0(bond_features, ex, W_i[:bdim], W_i[bdim:])
    si, bi = _scale_shift(st0, n_bonds, bn_i_g, bn_i_b)

    y = None
    sm = bm = None
    for _ in range(3):
        gext = _tc_msg_matmul(h0p, y, si, bi, sm, bm, W_m)
        a_tab = _sc_atom_sums(gext, gidx3, a_pad)
        y = _sc_messages(a_tab, gext, orig3, brm3, n_bonds)
        st = _tc_stats(y)
        sm, bm = _scale_shift(st, n_bonds, bn_m_g, bn_m_b)

    c1 = (jnp.asarray(num_bonds_per_atom) - n_bonds // n_atoms).astype(F32)
    cvec = c1 * jnp.sum(W_a[adim:], axis=0, keepdims=True)
    pre, sta = _tc_readout(h0p, y, si, bi, sm, bm,
                           atom_features, W_a[:adim], W_a[adim:], cvec)
    sa, ba = _scale_shift(sta, n_atoms, bn_a_g, bn_a_b)
    h = _tc_mol(pre, sa, ba, n_mol, n_atoms // n_mol)
    h = h + (jnp.asarray(num_atoms_per_mol) - n_atoms // n_mol).astype(F32)
    return jnp.concatenate([h, molecule_features], axis=1)
